# baseline (speedup 1.0000x reference)
import functools

import jax
import jax.numpy as jnp
from jax import lax
from jax.experimental import pallas as pl
from jax.experimental.pallas import tpu as pltpu
from jax.experimental.pallas import tpu_sc as plsc

B, N, T, D = 16, 64, 32, 128
R = N * B
GBLK = 16
NW = 32
PW = R // NW


def _sc_gather_body(x_flat_hbm, table_hbm, feat_hbm, idx_v, acc_v, sem):
    wid = lax.axis_index("s") * 2 + lax.axis_index("c")
    pltpu.sync_copy(x_flat_hbm.at[pl.ds(wid * (T * PW), T * PW)], idx_v)
    pltpu.async_copy(table_hbm.at[idx_v.at[pl.ds(0, PW)]], acc_v, sem).wait()
    for c in range(4):
        lo = 1 + 8 * c
        hi = min(T, lo + 8)
        descs = [
            pltpu.async_copy(table_hbm.at[idx_v.at[pl.ds(t * PW, PW)]],
                             acc_v, sem, add=True)
            for t in range(lo, hi)
        ]
        for d in descs:
            d.wait()
    pltpu.sync_copy(acc_v, feat_hbm.at[pl.ds(wid * PW, PW)])


def _gather_feat_sums(x_flat, table):
    mesh = plsc.VectorSubcoreMesh(core_axis_name="c", subcore_axis_name="s")
    return pl.kernel(
        _sc_gather_body,
        out_type=jax.ShapeDtypeStruct((R, D), jnp.float32),
        mesh=mesh,
        scratch_types=[
            pltpu.VMEM((T * PW,), jnp.int32),
            pltpu.VMEM((PW, D), jnp.float32),
            pltpu.SemaphoreType.DMA,
        ],
    )(x_flat, table)


def _tc_body(feat_ref, fd_row, fs_colt, fs_row, fd_colt, bd_row, bs_colt,
             wxf, whf, bf, wxb, whb, bb, out_ref,
             af, ab, abk, xgf, xgb, hc, hcb, basef, baseb):
    f32 = jnp.float32
    bf16 = jnp.bfloat16
    iota_d = lax.broadcasted_iota(jnp.int32, (N, 128), 0)
    iota_k = lax.broadcasted_iota(jnp.int32, (128, R), 1)

    for b in range(B):
        drows = (fd_row[b:b + 1, :] == iota_d).astype(f32)
        scols = (iota_k == fs_colt[:, b:b + 1] * B + b).astype(f32)
        af[:, b, :] = jnp.minimum(
            jax.lax.dot(drows, scols, precision=lax.Precision.DEFAULT,
                        preferred_element_type=f32), 1.0)

        drows = (bd_row[b:b + 1, :] == iota_d).astype(f32)
        scols = (iota_k == bs_colt[:, b:b + 1] * B + b).astype(f32)
        ab[:, b, :] = jnp.minimum(
            jax.lax.dot(drows, scols, precision=lax.Precision.DEFAULT,
                        preferred_element_type=f32), 1.0)

        drows = (fs_row[b:b + 1, :] == iota_d).astype(f32)
        scols = (iota_k == fd_colt[:, b:b + 1] * B + b).astype(f32)
        abk[:, b, :] = jnp.minimum(
            jax.lax.dot(drows, scols, precision=lax.Precision.DEFAULT,
                        preferred_element_type=f32), 1.0)

    featf = feat_ref[...] * (1.0 / T)
    xgf[...] = (jax.lax.dot(featf, wxf[...], preferred_element_type=f32)
                + bf[...]).reshape(N, B, 512)
    xgb[...] = (jax.lax.dot(featf, wxb[...], preferred_element_type=f32)
                + bb[...]).reshape(N, B, 512)

    hc[...] = jnp.zeros((N, B, 2 * D), f32)
    hcb[...] = jnp.zeros((N, B, 2 * D), f32)

    def gate_math(agg, z_x, wh_ref):
        aggh = agg[:, :D]
        aggc = agg[:, D:]
        z = z_x + jax.lax.dot(aggh, wh_ref[...],
                              precision=lax.Precision.DEFAULT,
                              preferred_element_type=f32)
        sg = 0.5 * jnp.tanh(0.5 * z[:, :2 * D]) + 0.5
        ig = sg[:, :D]
        og = sg[:, D:]
        ug = jnp.tanh(z[:, 2 * D:3 * D])
        fg = 0.5 * jnp.tanh(0.5 * z[:, 3 * D:]) + 0.5
        c_new = ig * ug + fg * aggc
        h_new = og * jnp.tanh(c_new)
        return h_new, c_new

    G = GBLK
    NB = N // G
    GB = G * B

    for g in range(NB):
        gb = NB - 1 - g
        if g == 0:
            basef[...] = jnp.zeros((G, B, 2 * D), f32)
            baseb[...] = jnp.zeros((G, B, 2 * D), f32)
        else:
            lhs = af[G * g:G * (g + 1), :, :GB * g].reshape(GB, GB * g)
            rhs = hc[:G * g].reshape(GB * g, 2 * D)
            basef[...] = jax.lax.dot(
                lhs, rhs, precision=lax.Precision.DEFAULT,
                preferred_element_type=f32).reshape(G, B, 2 * D)
            lhsb = abk[G * gb:G * (gb + 1), :, GB * (gb + 1):].reshape(
                GB, GB * g)
            rhsb = hcb[G * (gb + 1):].reshape(GB * g, 2 * D)
            baseb[...] = jax.lax.dot(
                lhsb, rhsb, precision=lax.Precision.DEFAULT,
                preferred_element_type=f32).reshape(G, B, 2 * D)

        def blk(t4, _):
          for u in range(16):
            t = t4 * 16 + u
            i = G * g + t
            a_sl = af[i, :, GB * g:GB * (g + 1)]
            blkst = hc[G * g:G * (g + 1)].reshape(GB, 2 * D)
            agg = basef[t] + jax.lax.dot(a_sl, blkst,
                                         precision=lax.Precision.DEFAULT,
                                         preferred_element_type=f32)
            h_new, c_new = gate_math(agg, xgf[i], whf)
            hc[i, :, :D] = h_new
            hc[i, :, D:] = c_new

            ib = G * gb + (G - 1 - t)
            a_slb = abk[ib, :, GB * gb:GB * (gb + 1)]
            blkstb = hcb[G * gb:G * (gb + 1)].reshape(GB, 2 * D)
            aggb = baseb[G - 1 - t] + jax.lax.dot(
                a_slb, blkstb, precision=lax.Precision.DEFAULT,
                preferred_element_type=f32)
            hb_new, cb_new = gate_math(aggb, xgb[ib], whb)
            hcb[ib, :, :D] = hb_new
            hcb[ib, :, D:] = cb_new
          return 0

        lax.fori_loop(0, G // 16, blk, 0)

    low_mask = (lax.broadcasted_iota(jnp.int32, (GB, GB), 0) // B >
                lax.broadcasted_iota(jnp.int32, (GB, GB), 1) // B
                ).astype(f32)
    for g in range(NB):
        lhs = ab[G * g:G * (g + 1)].reshape(GB, R)
        base = jax.lax.dot(lhs, hc[:, :, :D].reshape(R, D),
                           precision=lax.Precision.DEFAULT,
                           preferred_element_type=f32)
        lmat = ab[G * g:G * (g + 1), :, GB * g:GB * (g + 1)].reshape(
            GB, GB) * low_mask
        d = base + jax.lax.dot(lmat, base, precision=lax.Precision.DEFAULT, preferred_element_type=f32)
        l2 = jax.lax.dot(lmat, lmat, precision=lax.Precision.DEFAULT, preferred_element_type=f32)
        d = d + jax.lax.dot(l2, d, precision=lax.Precision.DEFAULT, preferred_element_type=f32)
        l4 = jax.lax.dot(l2, l2, precision=lax.Precision.DEFAULT, preferred_element_type=f32)
        d = d + jax.lax.dot(l4, d, precision=lax.Precision.DEFAULT, preferred_element_type=f32)
        l8 = jax.lax.dot(l4, l4, precision=lax.Precision.DEFAULT, preferred_element_type=f32)
        d = d + jax.lax.dot(l8, d, precision=lax.Precision.DEFAULT, preferred_element_type=f32)
        hc[G * g:G * (g + 1), :, :D] = (
            hc[G * g:G * (g + 1), :, :D] + d.reshape(G, B, D))

    out_ref[:, :, :D] = hc[:, :, :D]
    out_ref[:, :, D:] = hcb[:, :, :D]


def _tc_call(feat, fd_row, fs_colt, fs_row, fd_colt, bd_row, bs_colt,
             wxf, whf, bf, wxb, whb, bb):
    f32 = jnp.float32
    return pl.pallas_call(
        _tc_body,
        out_shape=jax.ShapeDtypeStruct((N, B, 2 * D), f32),
        scratch_shapes=[
            pltpu.VMEM((N, B, R), f32),
            pltpu.VMEM((N, B, R), f32),
            pltpu.VMEM((N, B, R), f32),
            pltpu.VMEM((N, B, 4 * D), f32),
            pltpu.VMEM((N, B, 4 * D), f32),
            pltpu.VMEM((N, B, 2 * D), f32),
            pltpu.VMEM((N, B, 2 * D), f32),
            pltpu.VMEM((GBLK, B, 2 * D), f32),
            pltpu.VMEM((GBLK, B, 2 * D), f32),
        ],
    )(feat, fd_row, fs_colt, fs_row, fd_colt, bd_row, bs_colt,
      wxf, whf, bf, wxb, whb, bb)


def kernel(x, f_edges, b_edges, table, g_ax_w, g_ax_b, g_ah_w, g_ah_b,
           g_fx_w, g_fx_b, g_fh_w, g_fh_b, bg_ax_w, bg_ax_b, bg_ah_w,
           bg_ah_b, bg_fx_w, bg_fx_b, bg_fh_w, bg_fh_b):
    x_tn = jnp.transpose(x.astype(jnp.int32), (2, 1, 0)).reshape(T, NW, PW)
    x_flat = jnp.transpose(x_tn, (1, 0, 2)).reshape(NW * T * PW)
    feat = _gather_feat_sums(x_flat, table)

    f_src = f_edges[..., 0].astype(jnp.int32)
    f_dst = f_edges[..., 1].astype(jnp.int32)
    b_src = b_edges[..., 0].astype(jnp.int32)
    b_dst = b_edges[..., 1].astype(jnp.int32)

    wxf = jnp.concatenate([g_ax_w, g_fx_w], axis=0).T
    whf = jnp.concatenate([g_ah_w, g_fh_w], axis=0).T
    bf = jnp.concatenate([g_ax_b + g_ah_b, g_fx_b + g_fh_b]).reshape(1, 512)
    wxb = jnp.concatenate([bg_ax_w, bg_fx_w], axis=0).T
    whb = jnp.concatenate([bg_ah_w, bg_fh_w], axis=0).T
    bb = jnp.concatenate([bg_ax_b + bg_ah_b, bg_fx_b + bg_fh_b]).reshape(1, 512)

    res = _tc_call(feat, f_dst, f_src.T, f_src, f_dst.T, b_dst, b_src.T,
                   wxf, whf, bf, wxb, whb, bb)
    return jnp.transpose(res, (1, 0, 2))


# speedup vs baseline: 1.0204x; 1.0204x over previous
import functools

import jax
import jax.numpy as jnp
from jax import lax
from jax.experimental import pallas as pl
from jax.experimental.pallas import tpu as pltpu
from jax.experimental.pallas import tpu_sc as plsc

B, N, T, D = 16, 64, 32, 128
R = N * B
GBLK = 16
NW = 32
PW = R // NW


def _sc_gather_body(x_flat_hbm, table_hbm, feat_hbm, idx_v, acc_v, sem):
    wid = lax.axis_index("s") * 2 + lax.axis_index("c")
    pltpu.sync_copy(x_flat_hbm.at[pl.ds(wid * (T * PW), T * PW)], idx_v)
    pltpu.async_copy(table_hbm.at[idx_v.at[pl.ds(0, PW)]], acc_v, sem).wait()
    for c in range(4):
        lo = 1 + 8 * c
        hi = min(T, lo + 8)
        descs = [
            pltpu.async_copy(table_hbm.at[idx_v.at[pl.ds(t * PW, PW)]],
                             acc_v, sem, add=True)
            for t in range(lo, hi)
        ]
        for d in descs:
            d.wait()
    pltpu.sync_copy(acc_v, feat_hbm.at[pl.ds(wid * PW, PW)])


def _gather_feat_sums(x_flat, table):
    mesh = plsc.VectorSubcoreMesh(core_axis_name="c", subcore_axis_name="s")
    return pl.kernel(
        _sc_gather_body,
        out_type=jax.ShapeDtypeStruct((R, D), jnp.float32),
        mesh=mesh,
        scratch_types=[
            pltpu.VMEM((T * PW,), jnp.int32),
            pltpu.VMEM((PW, D), jnp.float32),
            pltpu.SemaphoreType.DMA,
        ],
    )(x_flat, table)


def _tc_body(feat_ref, fd_row, fs_colt, fs_row, fd_colt, bd_row, bs_colt,
             wxf, whf, bf, wxb, whb, bb, out_ref,
             af, ab, abk, xgf, xgb, hc, hcb, basef, baseb):
    f32 = jnp.float32
    bf16 = jnp.bfloat16
    iota_d = lax.broadcasted_iota(jnp.int32, (N, 128), 0)
    iota_k = lax.broadcasted_iota(jnp.int32, (128, R), 1)

    for b in range(B):
        drows = (fd_row[b:b + 1, :] == iota_d).astype(f32)
        scols = (iota_k == fs_colt[:, b:b + 1] * B + b).astype(f32)
        af[:, b, :] = jnp.minimum(
            jax.lax.dot(drows, scols, precision=lax.Precision.DEFAULT,
                        preferred_element_type=f32), 1.0)

        drows = (bd_row[b:b + 1, :] == iota_d).astype(f32)
        scols = (iota_k == bs_colt[:, b:b + 1] * B + b).astype(f32)
        ab[:, b, :] = jnp.minimum(
            jax.lax.dot(drows, scols, precision=lax.Precision.DEFAULT,
                        preferred_element_type=f32), 1.0)

        drows = (fs_row[b:b + 1, :] == iota_d).astype(f32)
        scols = (iota_k == fd_colt[:, b:b + 1] * B + b).astype(f32)
        abk[:, b, :] = jnp.minimum(
            jax.lax.dot(drows, scols, precision=lax.Precision.DEFAULT,
                        preferred_element_type=f32), 1.0)

    featf = feat_ref[...] * (1.0 / T)
    xgf[...] = (jax.lax.dot(featf, wxf[...], preferred_element_type=f32)
                + bf[...]).reshape(N, B, 512)
    xgb[...] = (jax.lax.dot(featf, wxb[...], preferred_element_type=f32)
                + bb[...]).reshape(N, B, 512)

    hc[...] = jnp.zeros((N, B, 2 * D), f32)
    hcb[...] = jnp.zeros((N, B, 2 * D), f32)

    def gate_math(agg, z_x, wh_ref):
        aggh = agg[:, :D]
        aggc = agg[:, D:]
        z = z_x + jax.lax.dot(aggh, wh_ref[...],
                              precision=lax.Precision.DEFAULT,
                              preferred_element_type=f32)
        sg = 0.5 * jnp.tanh(0.5 * z[:, :2 * D]) + 0.5
        ig = sg[:, :D]
        og = sg[:, D:]
        ug = jnp.tanh(z[:, 2 * D:3 * D])
        fg = 0.5 * jnp.tanh(0.5 * z[:, 3 * D:]) + 0.5
        c_new = ig * ug + fg * aggc
        h_new = og * jnp.tanh(c_new)
        return h_new, c_new

    G = GBLK
    NB = N // G
    GB = G * B

    for g in range(NB):
        gb = NB - 1 - g
        if g == 0:
            basef[...] = jnp.zeros((G, B, 2 * D), f32)
            baseb[...] = jnp.zeros((G, B, 2 * D), f32)
        else:
            lhs = af[G * g:G * (g + 1), :, :GB * g].reshape(GB, GB * g)
            rhs = hc[:G * g].reshape(GB * g, 2 * D)
            basef[...] = jax.lax.dot(
                lhs, rhs, precision=lax.Precision.DEFAULT,
                preferred_element_type=f32).reshape(G, B, 2 * D)
            lhsb = abk[G * gb:G * (gb + 1), :, GB * (gb + 1):].reshape(
                GB, GB * g)
            rhsb = hcb[G * (gb + 1):].reshape(GB * g, 2 * D)
            baseb[...] = jax.lax.dot(
                lhsb, rhsb, precision=lax.Precision.DEFAULT,
                preferred_element_type=f32).reshape(G, B, 2 * D)

        pre_f = None
        pre_b = None
        for t in range(G):
            i = G * g + t
            if t == 0:
                agg = basef[0]
            else:
                a_last = af[i, :, GB * g + B * (t - 1):GB * g + B * t]
                agg = pre_f + jax.lax.dot(
                    a_last, hc[i - 1], precision=lax.Precision.DEFAULT,
                    preferred_element_type=f32)
            h_new, c_new = gate_math(agg, xgf[i], whf)
            hc[i, :, :D] = h_new
            hc[i, :, D:] = c_new

            ib = G * gb + (G - 1 - t)
            if t == 0:
                aggb = baseb[G - 1]
            else:
                a_lastb = abk[ib, :,
                              GB * gb + B * (G - t):GB * gb + B * (G + 1 - t)]
                aggb = pre_b + jax.lax.dot(
                    a_lastb, hcb[ib + 1], precision=lax.Precision.DEFAULT,
                    preferred_element_type=f32)
            hb_new, cb_new = gate_math(aggb, xgb[ib], whb)
            hcb[ib, :, :D] = hb_new
            hcb[ib, :, D:] = cb_new

            if t + 1 < G:
                if t == 0:
                    pre_f = basef[1]
                    pre_b = baseb[G - 2]
                else:
                    pre_f = basef[t + 1] + jax.lax.dot(
                        af[i + 1, :, GB * g:GB * g + B * t],
                        hc[G * g:G * g + t].reshape(B * t, 2 * D),
                        precision=lax.Precision.DEFAULT,
                        preferred_element_type=f32)
                    pre_b = baseb[G - 2 - t] + jax.lax.dot(
                        abk[ib - 1, :, GB * gb + B * (G - t):GB * (gb + 1)],
                        hcb[G * gb + G - t:G * (gb + 1)].reshape(B * t, 2 * D),
                        precision=lax.Precision.DEFAULT,
                        preferred_element_type=f32)

    low_mask = (lax.broadcasted_iota(jnp.int32, (GB, GB), 0) // B >
                lax.broadcasted_iota(jnp.int32, (GB, GB), 1) // B
                ).astype(f32)
    for g in range(NB):
        lhs = ab[G * g:G * (g + 1)].reshape(GB, R)
        base = jax.lax.dot(lhs, hc[:, :, :D].reshape(R, D),
                           precision=lax.Precision.DEFAULT,
                           preferred_element_type=f32)
        lmat = ab[G * g:G * (g + 1), :, GB * g:GB * (g + 1)].reshape(
            GB, GB) * low_mask
        d = base + jax.lax.dot(lmat, base, precision=lax.Precision.DEFAULT, preferred_element_type=f32)
        l2 = jax.lax.dot(lmat, lmat, precision=lax.Precision.DEFAULT, preferred_element_type=f32)
        d = d + jax.lax.dot(l2, d, precision=lax.Precision.DEFAULT, preferred_element_type=f32)
        l4 = jax.lax.dot(l2, l2, precision=lax.Precision.DEFAULT, preferred_element_type=f32)
        d = d + jax.lax.dot(l4, d, precision=lax.Precision.DEFAULT, preferred_element_type=f32)
        l8 = jax.lax.dot(l4, l4, precision=lax.Precision.DEFAULT, preferred_element_type=f32)
        d = d + jax.lax.dot(l8, d, precision=lax.Precision.DEFAULT, preferred_element_type=f32)
        hc[G * g:G * (g + 1), :, :D] = (
            hc[G * g:G * (g + 1), :, :D] + d.reshape(G, B, D))

    out_ref[:, :, :D] = hc[:, :, :D]
    out_ref[:, :, D:] = hcb[:, :, :D]


def _tc_call(feat, fd_row, fs_colt, fs_row, fd_colt, bd_row, bs_colt,
             wxf, whf, bf, wxb, whb, bb):
    f32 = jnp.float32
    return pl.pallas_call(
        _tc_body,
        out_shape=jax.ShapeDtypeStruct((N, B, 2 * D), f32),
        scratch_shapes=[
            pltpu.VMEM((N, B, R), f32),
            pltpu.VMEM((N, B, R), f32),
            pltpu.VMEM((N, B, R), f32),
            pltpu.VMEM((N, B, 4 * D), f32),
            pltpu.VMEM((N, B, 4 * D), f32),
            pltpu.VMEM((N, B, 2 * D), f32),
            pltpu.VMEM((N, B, 2 * D), f32),
            pltpu.VMEM((GBLK, B, 2 * D), f32),
            pltpu.VMEM((GBLK, B, 2 * D), f32),
        ],
    )(feat, fd_row, fs_colt, fs_row, fd_colt, bd_row, bs_colt,
      wxf, whf, bf, wxb, whb, bb)


def kernel(x, f_edges, b_edges, table, g_ax_w, g_ax_b, g_ah_w, g_ah_b,
           g_fx_w, g_fx_b, g_fh_w, g_fh_b, bg_ax_w, bg_ax_b, bg_ah_w,
           bg_ah_b, bg_fx_w, bg_fx_b, bg_fh_w, bg_fh_b):
    x_tn = jnp.transpose(x.astype(jnp.int32), (2, 1, 0)).reshape(T, NW, PW)
    x_flat = jnp.transpose(x_tn, (1, 0, 2)).reshape(NW * T * PW)
    feat = _gather_feat_sums(x_flat, table)

    f_src = f_edges[..., 0].astype(jnp.int32)
    f_dst = f_edges[..., 1].astype(jnp.int32)
    b_src = b_edges[..., 0].astype(jnp.int32)
    b_dst = b_edges[..., 1].astype(jnp.int32)

    wxf = jnp.concatenate([g_ax_w, g_fx_w], axis=0).T
    whf = jnp.concatenate([g_ah_w, g_fh_w], axis=0).T
    bf = jnp.concatenate([g_ax_b + g_ah_b, g_fx_b + g_fh_b]).reshape(1, 512)
    wxb = jnp.concatenate([bg_ax_w, bg_fx_w], axis=0).T
    whb = jnp.concatenate([bg_ah_w, bg_fh_w], axis=0).T
    bb = jnp.concatenate([bg_ax_b + bg_ah_b, bg_fx_b + bg_fh_b]).reshape(1, 512)

    res = _tc_call(feat, f_dst, f_src.T, f_src, f_dst.T, b_dst, b_src.T,
                   wxf, whf, bf, wxb, whb, bb)
    return jnp.transpose(res, (1, 0, 2))
